# baseline (speedup 1.0000x reference)
import functools

import jax
import jax.numpy as jnp
from jax import lax
from jax.experimental import pallas as pl
from jax.experimental.pallas import tpu as pltpu
from jax.experimental.pallas import tpu_sc as plsc

_NUM_EMBEDDINGS = 8192
_DIM = 256
_ROWS = 4096

_RB = 1024
_CB = 1024
_NR = _ROWS // _RB
_NC = _NUM_EMBEDDINGS // _CB
_CHUNKS = _CB // 128


def _argmin_body(e2b, xb, xn, out_idx, out_et, minv, mini, en_s):
    c = pl.program_id(0)
    r = pl.program_id(1)

    @pl.when(r == 0)
    def _per_col_block():
        e2 = e2b[...]
        en_s[...] = 0.25 * jnp.sum(e2 * e2, axis=0, keepdims=True)
        out_et[...] = -0.5 * e2.T

    rsl = pl.ds(r * _RB, _RB)

    @pl.when(c == 0)
    def _init():
        minv[rsl, :] = jnp.full((_RB, 128), jnp.inf, dtype=jnp.float32)
        mini[rsl, :] = jnp.zeros((_RB, 128), dtype=jnp.int32)

    s2 = jnp.dot(xb[...], e2b[...], preferred_element_type=jnp.float32)
    xn_v = xn[...]
    en_v = en_s[...]
    rm = minv[rsl, :]
    ri = mini[rsl, :]
    for k in range(_CHUNKS):
        sl = slice(k * 128, (k + 1) * 128)
        dk = (xn_v + en_v[:, sl]) + s2[:, sl]
        upd = dk < rm
        rm = jnp.where(upd, dk, rm)
        ri = jnp.where(upd, jnp.full((_RB, 128), c * _CHUNKS + k, jnp.int32), ri)
    minv[rsl, :] = rm
    mini[rsl, :] = ri

    @pl.when(c == pl.num_programs(0) - 1)
    def _extract():
        lane = lax.broadcasted_iota(jnp.int32, (_RB, 128), 1)
        idx = ri * 128 + lane
        m = jnp.min(rm, axis=1, keepdims=True)
        cand = jnp.where(rm == m, idx, jnp.int32(2**30))
        out_idx[...] = jnp.min(cand, axis=1, keepdims=True)


def _tc_argmin(flattened, emb_neg2, x_norm):
    out_idx, out_et = pl.pallas_call(
        _argmin_body,
        grid=(_NC, _NR),
        in_specs=[
            pl.BlockSpec((_DIM, _CB), lambda c, r: (0, c)),
            pl.BlockSpec((_RB, _DIM), lambda c, r: (r, 0)),
            pl.BlockSpec((_RB, 1), lambda c, r: (r, 0)),
        ],
        out_specs=[
            pl.BlockSpec((_RB, 1), lambda c, r: (r, 0)),
            pl.BlockSpec((_CB, _DIM), lambda c, r: (c, 0)),
        ],
        out_shape=[
            jax.ShapeDtypeStruct((_ROWS, 1), jnp.int32),
            jax.ShapeDtypeStruct((_NUM_EMBEDDINGS, _DIM), jnp.float32),
        ],
        scratch_shapes=[
            pltpu.VMEM((_ROWS, 128), jnp.float32),
            pltpu.VMEM((_ROWS, 128), jnp.int32),
            pltpu.VMEM((1, _CB), jnp.float32),
        ],
    )(emb_neg2, flattened, x_norm)
    return out_idx.reshape(_ROWS), out_et


def _sc_gather(table, idx):
    info = plsc.get_sparse_core_info()
    ncores, nsub = info.num_cores, info.num_subcores
    nw = ncores * nsub
    b_per_w = _ROWS // nw
    mesh = plsc.VectorSubcoreMesh(core_axis_name="c", subcore_axis_name="s")

    @functools.partial(
        pl.kernel,
        mesh=mesh,
        out_type=jax.ShapeDtypeStruct((_ROWS, _DIM), jnp.float32),
        scratch_types=[
            pltpu.VMEM((b_per_w,), jnp.int32),
            pltpu.VMEM((b_per_w, _DIM), jnp.float32),
            pltpu.SemaphoreType.DMA,
        ],
    )
    def gk(table_hbm, idx_hbm, out_hbm, idx_v, rows_v, sem):
        wid = lax.axis_index("s") * ncores + lax.axis_index("c")
        base = wid * b_per_w
        pltpu.sync_copy(idx_hbm.at[pl.ds(base, b_per_w)], idx_v)
        pltpu.async_copy(table_hbm.at[idx_v], rows_v, sem).wait()
        pltpu.sync_copy(rows_v, out_hbm.at[pl.ds(base, b_per_w)])

    return gk(table, idx)


def kernel(x, embeddings):
    input_shape = x.shape
    flattened = jnp.reshape(x, (-1, _DIM))
    x_norm = jnp.sum(flattened**2, axis=1, keepdims=True)

    idx, emb_t = _tc_argmin(flattened, -2.0 * embeddings, x_norm)

    quantized = _sc_gather(emb_t, idx)
    return (jnp.reshape(quantized, input_shape), idx)


# speedup vs baseline: 1.0400x; 1.0400x over previous
import functools

import jax
import jax.numpy as jnp
from jax import lax
from jax.experimental import pallas as pl
from jax.experimental.pallas import tpu as pltpu
from jax.experimental.pallas import tpu_sc as plsc

_NUM_EMBEDDINGS = 8192
_DIM = 256
_ROWS = 4096

_RB = 1024
_CB = 1024
_NR = _ROWS // _RB
_NC = _NUM_EMBEDDINGS // _CB
_MM = 256


def _prep_body(eb, e2, et, en):
    e = eb[...]
    e2[...] = -2.0 * e
    et[...] = e.T
    en[...] = jnp.sum(e * e, axis=0, keepdims=True)


def _prep(embeddings):
    return pl.pallas_call(
        _prep_body,
        grid=(_NC,),
        in_specs=[pl.BlockSpec((_DIM, _CB), lambda c: (0, c))],
        out_specs=[
            pl.BlockSpec((_DIM, _CB), lambda c: (0, c)),
            pl.BlockSpec((_CB, _DIM), lambda c: (c, 0)),
            pl.BlockSpec((1, _CB), lambda c: (0, c)),
        ],
        out_shape=[
            jax.ShapeDtypeStruct((_DIM, _NUM_EMBEDDINGS), jnp.float32),
            jax.ShapeDtypeStruct((_NUM_EMBEDDINGS, _DIM), jnp.float32),
            jax.ShapeDtypeStruct((1, _NUM_EMBEDDINGS), jnp.float32),
        ],
    )(embeddings)


def _argmin_body(xb, e2b, xn, en, rm_ref, ri_ref):
    c = pl.program_id(1)

    @pl.when(c == 0)
    def _init():
        rm_ref[...] = jnp.full((_RB, 128), jnp.inf, dtype=jnp.float32)
        ri_ref[...] = jnp.zeros((_RB, 128), dtype=jnp.int32)

    xb_v = xb[...]
    xn_v = xn[...]
    en_v = en[...]
    rm = rm_ref[...]
    ri = ri_ref[...]
    for km in range(_CB // _MM):
        s2 = jnp.dot(
            xb_v,
            e2b[:, km * _MM : (km + 1) * _MM],
            preferred_element_type=jnp.float32,
        )
        for kk in range(_MM // 128):
            k = km * (_MM // 128) + kk
            sl = slice(k * 128, (k + 1) * 128)
            dk = (xn_v + en_v[:, sl]) + s2[:, kk * 128 : (kk + 1) * 128]
            upd = dk < rm
            rm = jnp.where(upd, dk, rm)
            ri = jnp.where(
                upd, jnp.full((_RB, 128), c * (_CB // 128) + k, jnp.int32), ri
            )
    rm_ref[...] = rm
    ri_ref[...] = ri


def _tc_argmin(flattened, emb_neg2, x_norm, e_norm):
    return pl.pallas_call(
        _argmin_body,
        grid=(_NR, _NC),
        in_specs=[
            pl.BlockSpec((_RB, _DIM), lambda r, c: (r, 0)),
            pl.BlockSpec((_DIM, _CB), lambda r, c: (0, c)),
            pl.BlockSpec((_RB, 1), lambda r, c: (r, 0)),
            pl.BlockSpec((1, _CB), lambda r, c: (0, c)),
        ],
        out_specs=[
            pl.BlockSpec((_RB, 128), lambda r, c: (r, 0)),
            pl.BlockSpec((_RB, 128), lambda r, c: (r, 0)),
        ],
        out_shape=[
            jax.ShapeDtypeStruct((_ROWS, 128), jnp.float32),
            jax.ShapeDtypeStruct((_ROWS, 128), jnp.int32),
        ],
    )(flattened, emb_neg2, x_norm, e_norm)


def _extract_body(rm_b, ri_b, out_idx):
    rm = rm_b[...]
    lane = lax.broadcasted_iota(jnp.int32, (_RB, 128), 1)
    idx = ri_b[...] * 128 + lane
    m = jnp.min(rm, axis=1, keepdims=True)
    cand = jnp.where(rm == m, idx, jnp.int32(2**30))
    out_idx[...] = jnp.min(cand, axis=1, keepdims=True)


def _extract(rm, ri):
    out = pl.pallas_call(
        _extract_body,
        grid=(_NR,),
        in_specs=[
            pl.BlockSpec((_RB, 128), lambda r: (r, 0)),
            pl.BlockSpec((_RB, 128), lambda r: (r, 0)),
        ],
        out_specs=pl.BlockSpec((_RB, 1), lambda r: (r, 0)),
        out_shape=jax.ShapeDtypeStruct((_ROWS, 1), jnp.int32),
    )(rm, ri)
    return out.reshape(_ROWS)


def _sc_gather(table, idx):
    info = plsc.get_sparse_core_info()
    ncores, nsub = info.num_cores, info.num_subcores
    nw = ncores * nsub
    b_per_w = _ROWS // nw
    mesh = plsc.VectorSubcoreMesh(core_axis_name="c", subcore_axis_name="s")

    @functools.partial(
        pl.kernel,
        mesh=mesh,
        out_type=jax.ShapeDtypeStruct((_ROWS, _DIM), jnp.float32),
        scratch_types=[
            pltpu.VMEM((b_per_w,), jnp.int32),
            pltpu.VMEM((b_per_w, _DIM), jnp.float32),
            pltpu.SemaphoreType.DMA,
        ],
    )
    def gk(table_hbm, idx_hbm, out_hbm, idx_v, rows_v, sem):
        wid = lax.axis_index("s") * ncores + lax.axis_index("c")
        base = wid * b_per_w
        pltpu.sync_copy(idx_hbm.at[pl.ds(base, b_per_w)], idx_v)
        pltpu.async_copy(table_hbm.at[idx_v], rows_v, sem).wait()
        pltpu.sync_copy(rows_v, out_hbm.at[pl.ds(base, b_per_w)])

    return gk(table, idx)


def kernel(x, embeddings):
    input_shape = x.shape
    flattened = jnp.reshape(x, (-1, _DIM))
    x_norm = jnp.sum(flattened**2, axis=1, keepdims=True)

    emb_neg2, emb_t, e_norm = _prep(embeddings)
    rm, ri = _tc_argmin(flattened, emb_neg2, x_norm, e_norm)
    idx = _extract(rm, ri)

    quantized = _sc_gather(emb_t, idx)
    return (jnp.reshape(quantized, input_shape), idx)


# speedup vs baseline: 1.1592x; 1.1147x over previous
import functools

import jax
import jax.numpy as jnp
from jax import lax
from jax.experimental import pallas as pl
from jax.experimental.pallas import tpu as pltpu
from jax.experimental.pallas import tpu_sc as plsc

_NUM_EMBEDDINGS = 8192
_DIM = 256
_ROWS = 4096

_RB = 1024
_CB = 1024
_NR = _ROWS // _RB
_NC = _NUM_EMBEDDINGS // _CB
_MM = 256


def _vq_body(x_full, e_full, xn_full, out_et, out_idx, e2s, ens, rm_s, ri_s):
    c = pl.program_id(0)
    r = pl.program_id(1)

    @pl.when((c == 0) & (r == 0))
    def _derive():
        e = e_full[...]
        e2s[...] = -2.0 * e
        ens[...] = jnp.sum(e * e, axis=0, keepdims=True)

    @pl.when(r == 0)
    def _transpose():
        out_et[...] = e_full[:, pl.ds(c * _CB, _CB)].T

    rsl = pl.ds(r * _RB, _RB)

    @pl.when(c == 0)
    def _init():
        rm_s[rsl, :] = jnp.full((_RB, 128), jnp.inf, dtype=jnp.float32)
        ri_s[rsl, :] = jnp.zeros((_RB, 128), dtype=jnp.int32)

    xb = x_full[rsl, :]
    xn_v = xn_full[rsl, :]
    rm = rm_s[rsl, :]
    ri = ri_s[rsl, :]
    for km in range(_CB // _MM):
        s2 = jnp.dot(
            xb,
            e2s[:, pl.ds(c * _CB + km * _MM, _MM)],
            preferred_element_type=jnp.float32,
        )
        for kk in range(_MM // 128):
            k = km * (_MM // 128) + kk
            en_sl = ens[:, pl.ds(c * _CB + k * 128, 128)]
            dk = (xn_v + en_sl) + s2[:, kk * 128 : (kk + 1) * 128]
            upd = dk < rm
            rm = jnp.where(upd, dk, rm)
            ri = jnp.where(
                upd, jnp.full((_RB, 128), c * (_CB // 128) + k, jnp.int32), ri
            )
    rm_s[rsl, :] = rm
    ri_s[rsl, :] = ri

    @pl.when(c == pl.num_programs(0) - 1)
    def _extract():
        lane = lax.broadcasted_iota(jnp.int32, (_RB, 128), 1)
        idx = ri * 128 + lane
        m = jnp.min(rm, axis=1, keepdims=True)
        cand = jnp.where(rm == m, idx, jnp.int32(2**30))
        out_idx[...] = jnp.min(cand, axis=1, keepdims=True)


def _tc_vq(flattened, embeddings, x_norm):
    out_et, out_idx = pl.pallas_call(
        _vq_body,
        grid=(_NC, _NR),
        in_specs=[
            pl.BlockSpec((_ROWS, _DIM), lambda c, r: (0, 0)),
            pl.BlockSpec((_DIM, _NUM_EMBEDDINGS), lambda c, r: (0, 0)),
            pl.BlockSpec((_ROWS, 1), lambda c, r: (0, 0)),
        ],
        out_specs=[
            pl.BlockSpec((_CB, _DIM), lambda c, r: (c, 0)),
            pl.BlockSpec((_RB, 1), lambda c, r: (r, 0)),
        ],
        out_shape=[
            jax.ShapeDtypeStruct((_NUM_EMBEDDINGS, _DIM), jnp.float32),
            jax.ShapeDtypeStruct((_ROWS, 1), jnp.int32),
        ],
        scratch_shapes=[
            pltpu.VMEM((_DIM, _NUM_EMBEDDINGS), jnp.float32),
            pltpu.VMEM((1, _NUM_EMBEDDINGS), jnp.float32),
            pltpu.VMEM((_ROWS, 128), jnp.float32),
            pltpu.VMEM((_ROWS, 128), jnp.int32),
        ],
    )(flattened, embeddings, x_norm)
    return out_et, out_idx.reshape(_ROWS)


def _sc_gather(table, idx):
    info = plsc.get_sparse_core_info()
    ncores, nsub = info.num_cores, info.num_subcores
    nw = ncores * nsub
    b_per_w = _ROWS // nw
    mesh = plsc.VectorSubcoreMesh(core_axis_name="c", subcore_axis_name="s")

    @functools.partial(
        pl.kernel,
        mesh=mesh,
        out_type=jax.ShapeDtypeStruct((_ROWS, _DIM), jnp.float32),
        scratch_types=[
            pltpu.VMEM((b_per_w,), jnp.int32),
            pltpu.VMEM((b_per_w, _DIM), jnp.float32),
            pltpu.SemaphoreType.DMA,
        ],
    )
    def gk(table_hbm, idx_hbm, out_hbm, idx_v, rows_v, sem):
        wid = lax.axis_index("s") * ncores + lax.axis_index("c")
        base = wid * b_per_w
        pltpu.sync_copy(idx_hbm.at[pl.ds(base, b_per_w)], idx_v)
        pltpu.async_copy(table_hbm.at[idx_v], rows_v, sem).wait()
        pltpu.sync_copy(rows_v, out_hbm.at[pl.ds(base, b_per_w)])

    return gk(table, idx)


def kernel(x, embeddings):
    input_shape = x.shape
    flattened = jnp.reshape(x, (-1, _DIM))
    x_norm = jnp.sum(flattened**2, axis=1, keepdims=True)

    emb_t, idx = _tc_vq(flattened, embeddings, x_norm)

    quantized = _sc_gather(emb_t, idx)
    return (jnp.reshape(quantized, input_shape), idx)


# speedup vs baseline: 1.2529x; 1.0808x over previous
import functools

import jax
import jax.numpy as jnp
from jax import lax
from jax.experimental import pallas as pl
from jax.experimental.pallas import tpu as pltpu
from jax.experimental.pallas import tpu_sc as plsc

_NUM_EMBEDDINGS = 8192
_DIM = 256
_ROWS = 4096

_RB = 2048
_CB = 2048
_NR = _ROWS // _RB
_NC = _NUM_EMBEDDINGS // _CB
_MM = 256


def _vq_body(x_full, e_full, xn_full, out_et, out_idx, e2s, ens, rm_s, ri_s):
    c = pl.program_id(0)
    r = pl.program_id(1)

    @pl.when((c == 0) & (r == 0))
    def _derive():
        e = e_full[...]
        e2s[...] = -2.0 * e
        ens[...] = jnp.sum(e * e, axis=0, keepdims=True)

    @pl.when(r == 0)
    def _transpose():
        out_et[...] = e_full[:, pl.ds(c * _CB, _CB)].T

    rsl = pl.ds(r * _RB, _RB)

    @pl.when(c == 0)
    def _init():
        rm_s[rsl, :] = jnp.full((_RB, 128), jnp.inf, dtype=jnp.float32)
        ri_s[rsl, :] = jnp.zeros((_RB, 128), dtype=jnp.int32)

    xb = x_full[rsl, :]
    xn_v = xn_full[rsl, :]
    rm = rm_s[rsl, :]
    ri = ri_s[rsl, :]
    for km in range(_CB // _MM):
        s2 = jnp.dot(
            xb,
            e2s[:, pl.ds(c * _CB + km * _MM, _MM)],
            preferred_element_type=jnp.float32,
        )
        for kk in range(_MM // 128):
            k = km * (_MM // 128) + kk
            en_sl = ens[:, pl.ds(c * _CB + k * 128, 128)]
            dk = (xn_v + en_sl) + s2[:, kk * 128 : (kk + 1) * 128]
            upd = dk < rm
            rm = jnp.where(upd, dk, rm)
            ri = jnp.where(
                upd, jnp.full((_RB, 128), c * (_CB // 128) + k, jnp.int32), ri
            )
    rm_s[rsl, :] = rm
    ri_s[rsl, :] = ri

    @pl.when(c == pl.num_programs(0) - 1)
    def _extract():
        lane = lax.broadcasted_iota(jnp.int32, (_RB, 128), 1)
        idx = ri * 128 + lane
        m = jnp.min(rm, axis=1, keepdims=True)
        cand = jnp.where(rm == m, idx, jnp.int32(2**30))
        out_idx[...] = jnp.min(cand, axis=1, keepdims=True)


def _tc_vq(flattened, embeddings, x_norm):
    out_et, out_idx = pl.pallas_call(
        _vq_body,
        grid=(_NC, _NR),
        in_specs=[
            pl.BlockSpec((_ROWS, _DIM), lambda c, r: (0, 0)),
            pl.BlockSpec((_DIM, _NUM_EMBEDDINGS), lambda c, r: (0, 0)),
            pl.BlockSpec((_ROWS, 1), lambda c, r: (0, 0)),
        ],
        out_specs=[
            pl.BlockSpec((_CB, _DIM), lambda c, r: (c, 0)),
            pl.BlockSpec((_RB, 1), lambda c, r: (r, 0)),
        ],
        out_shape=[
            jax.ShapeDtypeStruct((_NUM_EMBEDDINGS, _DIM), jnp.float32),
            jax.ShapeDtypeStruct((_ROWS, 1), jnp.int32),
        ],
        scratch_shapes=[
            pltpu.VMEM((_DIM, _NUM_EMBEDDINGS), jnp.float32),
            pltpu.VMEM((1, _NUM_EMBEDDINGS), jnp.float32),
            pltpu.VMEM((_ROWS, 128), jnp.float32),
            pltpu.VMEM((_ROWS, 128), jnp.int32),
        ],
    )(flattened, embeddings, x_norm)
    return out_et, out_idx.reshape(_ROWS)


def _sc_gather(table, idx):
    info = plsc.get_sparse_core_info()
    ncores, nsub = info.num_cores, info.num_subcores
    nw = ncores * nsub
    b_per_w = _ROWS // nw
    mesh = plsc.VectorSubcoreMesh(core_axis_name="c", subcore_axis_name="s")

    @functools.partial(
        pl.kernel,
        mesh=mesh,
        out_type=jax.ShapeDtypeStruct((_ROWS, _DIM), jnp.float32),
        scratch_types=[
            pltpu.VMEM((b_per_w,), jnp.int32),
            pltpu.VMEM((b_per_w, _DIM), jnp.float32),
            pltpu.SemaphoreType.DMA,
        ],
    )
    def gk(table_hbm, idx_hbm, out_hbm, idx_v, rows_v, sem):
        wid = lax.axis_index("s") * ncores + lax.axis_index("c")
        base = wid * b_per_w
        pltpu.sync_copy(idx_hbm.at[pl.ds(base, b_per_w)], idx_v)
        pltpu.async_copy(table_hbm.at[idx_v], rows_v, sem).wait()
        pltpu.sync_copy(rows_v, out_hbm.at[pl.ds(base, b_per_w)])

    return gk(table, idx)


def kernel(x, embeddings):
    input_shape = x.shape
    flattened = jnp.reshape(x, (-1, _DIM))
    x_norm = jnp.sum(flattened**2, axis=1, keepdims=True)

    emb_t, idx = _tc_vq(flattened, embeddings, x_norm)

    quantized = _sc_gather(emb_t, idx)
    return (jnp.reshape(quantized, input_shape), idx)


# speedup vs baseline: 1.2564x; 1.0027x over previous
import functools

import jax
import jax.numpy as jnp
from jax import lax
from jax.experimental import pallas as pl
from jax.experimental.pallas import tpu as pltpu
from jax.experimental.pallas import tpu_sc as plsc

_NUM_EMBEDDINGS = 8192
_DIM = 256
_ROWS = 4096

_RB = 2048
_CB = 2048
_NR = _ROWS // _RB
_NC = _NUM_EMBEDDINGS // _CB
_MM = 512


def _vq_body(x_full, e_full, xn_full, out_et, out_idx, e2s, ens, rm_s, ri_s):
    c = pl.program_id(0)
    r = pl.program_id(1)

    @pl.when((c == 0) & (r == 0))
    def _derive():
        e = e_full[...]
        e2s[...] = -2.0 * e
        ens[...] = jnp.sum(e * e, axis=0, keepdims=True)

    @pl.when(r == 0)
    def _transpose():
        out_et[...] = e_full[:, pl.ds(c * _CB, _CB)].T

    rsl = pl.ds(r * _RB, _RB)

    @pl.when(c == 0)
    def _init():
        rm_s[rsl, :] = jnp.full((_RB, 128), jnp.inf, dtype=jnp.float32)
        ri_s[rsl, :] = jnp.zeros((_RB, 128), dtype=jnp.int32)

    xb = x_full[rsl, :]
    xn_v = xn_full[rsl, :]
    rm = rm_s[rsl, :]
    ri = ri_s[rsl, :]
    for km in range(_CB // _MM):
        s2 = jnp.dot(
            xb,
            e2s[:, pl.ds(c * _CB + km * _MM, _MM)],
            preferred_element_type=jnp.float32,
        )
        for kk in range(_MM // 128):
            k = km * (_MM // 128) + kk
            en_sl = ens[:, pl.ds(c * _CB + k * 128, 128)]
            dk = (xn_v + en_sl) + s2[:, kk * 128 : (kk + 1) * 128]
            upd = dk < rm
            rm = jnp.where(upd, dk, rm)
            ri = jnp.where(
                upd, jnp.full((_RB, 128), c * (_CB // 128) + k, jnp.int32), ri
            )
    rm_s[rsl, :] = rm
    ri_s[rsl, :] = ri

    @pl.when(c == pl.num_programs(0) - 1)
    def _extract():
        lane = lax.broadcasted_iota(jnp.int32, (_RB, 128), 1)
        idx = ri * 128 + lane
        m = jnp.min(rm, axis=1, keepdims=True)
        cand = jnp.where(rm == m, idx, jnp.int32(2**30))
        out_idx[...] = jnp.min(cand, axis=1, keepdims=True)


def _tc_vq(flattened, embeddings, x_norm):
    out_et, out_idx = pl.pallas_call(
        _vq_body,
        grid=(_NC, _NR),
        in_specs=[
            pl.BlockSpec((_ROWS, _DIM), lambda c, r: (0, 0)),
            pl.BlockSpec((_DIM, _NUM_EMBEDDINGS), lambda c, r: (0, 0)),
            pl.BlockSpec((_ROWS, 1), lambda c, r: (0, 0)),
        ],
        out_specs=[
            pl.BlockSpec((_CB, _DIM), lambda c, r: (c, 0)),
            pl.BlockSpec((_RB, 1), lambda c, r: (r, 0)),
        ],
        out_shape=[
            jax.ShapeDtypeStruct((_NUM_EMBEDDINGS, _DIM), jnp.float32),
            jax.ShapeDtypeStruct((_ROWS, 1), jnp.int32),
        ],
        scratch_shapes=[
            pltpu.VMEM((_DIM, _NUM_EMBEDDINGS), jnp.float32),
            pltpu.VMEM((1, _NUM_EMBEDDINGS), jnp.float32),
            pltpu.VMEM((_ROWS, 128), jnp.float32),
            pltpu.VMEM((_ROWS, 128), jnp.int32),
        ],
    )(flattened, embeddings, x_norm)
    return out_et, out_idx.reshape(_ROWS)


def _sc_gather(table, idx):
    info = plsc.get_sparse_core_info()
    ncores, nsub = info.num_cores, info.num_subcores
    nw = ncores * nsub
    b_per_w = _ROWS // nw
    mesh = plsc.VectorSubcoreMesh(core_axis_name="c", subcore_axis_name="s")

    @functools.partial(
        pl.kernel,
        mesh=mesh,
        out_type=jax.ShapeDtypeStruct((_ROWS, _DIM), jnp.float32),
        scratch_types=[
            pltpu.VMEM((b_per_w,), jnp.int32),
            pltpu.VMEM((b_per_w, _DIM), jnp.float32),
            pltpu.SemaphoreType.DMA,
        ],
    )
    def gk(table_hbm, idx_hbm, out_hbm, idx_v, rows_v, sem):
        wid = lax.axis_index("s") * ncores + lax.axis_index("c")
        base = wid * b_per_w
        pltpu.sync_copy(idx_hbm.at[pl.ds(base, b_per_w)], idx_v)
        pltpu.async_copy(table_hbm.at[idx_v], rows_v, sem).wait()
        pltpu.sync_copy(rows_v, out_hbm.at[pl.ds(base, b_per_w)])

    return gk(table, idx)


def kernel(x, embeddings):
    input_shape = x.shape
    flattened = jnp.reshape(x, (-1, _DIM))
    x_norm = jnp.sum(flattened**2, axis=1, keepdims=True)

    emb_t, idx = _tc_vq(flattened, embeddings, x_norm)

    quantized = _sc_gather(emb_t, idx)
    return (jnp.reshape(quantized, input_shape), idx)


# speedup vs baseline: 1.2875x; 1.0248x over previous
import functools

import jax
import jax.numpy as jnp
from jax import lax
from jax.experimental import pallas as pl
from jax.experimental.pallas import tpu as pltpu
from jax.experimental.pallas import tpu_sc as plsc

_NUM_EMBEDDINGS = 8192
_DIM = 256
_ROWS = 4096

_RB = 4096
_CB = 2048
_NR = _ROWS // _RB
_NC = _NUM_EMBEDDINGS // _CB
_MM = 512


def _vq_body(x_full, e_full, xn_full, out_et, out_idx, e2s, ens, rm_s, ri_s):
    c = pl.program_id(0)
    r = pl.program_id(1)

    @pl.when((c == 0) & (r == 0))
    def _derive():
        e = e_full[...]
        e2s[...] = -2.0 * e
        ens[...] = jnp.sum(e * e, axis=0, keepdims=True)

    @pl.when(r == 0)
    def _transpose():
        out_et[...] = e_full[:, pl.ds(c * _CB, _CB)].T

    rsl = pl.ds(r * _RB, _RB)

    @pl.when(c == 0)
    def _init():
        rm_s[rsl, :] = jnp.full((_RB, 128), jnp.inf, dtype=jnp.float32)
        ri_s[rsl, :] = jnp.zeros((_RB, 128), dtype=jnp.int32)

    xb = x_full[rsl, :]
    xn_v = xn_full[rsl, :]
    rm = rm_s[rsl, :]
    ri = ri_s[rsl, :]
    for km in range(_CB // _MM):
        s2 = jnp.dot(
            xb,
            e2s[:, pl.ds(c * _CB + km * _MM, _MM)],
            preferred_element_type=jnp.float32,
        )
        for kk in range(_MM // 128):
            k = km * (_MM // 128) + kk
            en_sl = ens[:, pl.ds(c * _CB + k * 128, 128)]
            dk = (xn_v + en_sl) + s2[:, kk * 128 : (kk + 1) * 128]
            upd = dk < rm
            rm = jnp.where(upd, dk, rm)
            ri = jnp.where(
                upd, jnp.full((_RB, 128), c * (_CB // 128) + k, jnp.int32), ri
            )
    rm_s[rsl, :] = rm
    ri_s[rsl, :] = ri

    @pl.when(c == pl.num_programs(0) - 1)
    def _extract():
        lane = lax.broadcasted_iota(jnp.int32, (_RB, 128), 1)
        idx = ri * 128 + lane
        m = jnp.min(rm, axis=1, keepdims=True)
        cand = jnp.where(rm == m, idx, jnp.int32(2**30))
        out_idx[...] = jnp.min(cand, axis=1, keepdims=True)


def _tc_vq(flattened, embeddings, x_norm):
    out_et, out_idx = pl.pallas_call(
        _vq_body,
        grid=(_NC, _NR),
        in_specs=[
            pl.BlockSpec((_ROWS, _DIM), lambda c, r: (0, 0)),
            pl.BlockSpec((_DIM, _NUM_EMBEDDINGS), lambda c, r: (0, 0)),
            pl.BlockSpec((_ROWS, 1), lambda c, r: (0, 0)),
        ],
        out_specs=[
            pl.BlockSpec((_CB, _DIM), lambda c, r: (c, 0)),
            pl.BlockSpec((_RB, 1), lambda c, r: (r, 0)),
        ],
        out_shape=[
            jax.ShapeDtypeStruct((_NUM_EMBEDDINGS, _DIM), jnp.float32),
            jax.ShapeDtypeStruct((_ROWS, 1), jnp.int32),
        ],
        scratch_shapes=[
            pltpu.VMEM((_DIM, _NUM_EMBEDDINGS), jnp.float32),
            pltpu.VMEM((1, _NUM_EMBEDDINGS), jnp.float32),
            pltpu.VMEM((_ROWS, 128), jnp.float32),
            pltpu.VMEM((_ROWS, 128), jnp.int32),
        ],
    )(flattened, embeddings, x_norm)
    return out_et, out_idx.reshape(_ROWS)


def _sc_gather(table, idx):
    info = plsc.get_sparse_core_info()
    ncores, nsub = info.num_cores, info.num_subcores
    nw = ncores * nsub
    b_per_w = _ROWS // nw
    mesh = plsc.VectorSubcoreMesh(core_axis_name="c", subcore_axis_name="s")

    @functools.partial(
        pl.kernel,
        mesh=mesh,
        out_type=jax.ShapeDtypeStruct((_ROWS, _DIM), jnp.float32),
        scratch_types=[
            pltpu.VMEM((b_per_w,), jnp.int32),
            pltpu.VMEM((b_per_w, _DIM), jnp.float32),
            pltpu.SemaphoreType.DMA,
        ],
    )
    def gk(table_hbm, idx_hbm, out_hbm, idx_v, rows_v, sem):
        wid = lax.axis_index("s") * ncores + lax.axis_index("c")
        base = wid * b_per_w
        pltpu.sync_copy(idx_hbm.at[pl.ds(base, b_per_w)], idx_v)
        pltpu.async_copy(table_hbm.at[idx_v], rows_v, sem).wait()
        pltpu.sync_copy(rows_v, out_hbm.at[pl.ds(base, b_per_w)])

    return gk(table, idx)


def kernel(x, embeddings):
    input_shape = x.shape
    flattened = jnp.reshape(x, (-1, _DIM))
    x_norm = jnp.sum(flattened**2, axis=1, keepdims=True)

    emb_t, idx = _tc_vq(flattened, embeddings, x_norm)

    quantized = _sc_gather(emb_t, idx)
    return (jnp.reshape(quantized, input_shape), idx)
